# baseline (speedup 1.0000x reference)
import jax
import jax.numpy as jnp
from jax import lax
from jax.experimental import pallas as pl
from jax.experimental.pallas import tpu as pltpu
from jax.experimental.pallas import tpu_sc as plsc

CHAR_VOCAB = 1000
EMBED_DIM = 64
BATCH = 1024
MAX_WORDS = 50
MAX_CHARS = 16

PAIRS = EMBED_DIM // 2
NUM_WORDS = BATCH * MAX_WORDS
NUM_WORKERS = 32
WORDS_PER_WORKER = NUM_WORDS // NUM_WORKERS
CHARS_PER_WORKER = WORDS_PER_WORKER * MAX_CHARS
OUT_PER_WORKER = WORDS_PER_WORKER * PAIRS
DGROUPS = PAIRS // 16


def _sc_body(chars_hbm, table_hbm, out_hbm, table_v, chars_v, out_v):
    wid = lax.axis_index("s") * 2 + lax.axis_index("c")

    pltpu.sync_copy(table_hbm, table_v)
    pltpu.sync_copy(chars_hbm.at[pl.ds(wid * CHARS_PER_WORKER, CHARS_PER_WORKER)],
                    chars_v)

    lanes = lax.iota(jnp.int32, 16)
    lanes_g = [lanes + 16 * g for g in range(DGROUPS)]

    @plsc.parallel_loop(0, WORDS_PER_WORKER, unroll=4)
    def word_body(w):
        cvec = chars_v[pl.ds(w * MAX_CHARS, MAX_CHARS)] * PAIRS
        ob = w * PAIRS
        for g in range(DGROUPS):
            vals = [plsc.bitcast(
                        plsc.load_gather(table_v, [lanes_g[g] + cvec[c]]),
                        jnp.bfloat16)
                    for c in range(MAX_CHARS)]
            while len(vals) > 1:
                vals = [jnp.maximum(vals[i], vals[i + 1])
                        for i in range(0, len(vals), 2)]
            plsc.store_scatter(out_v, [lanes_g[g] + ob],
                               plsc.bitcast(vals[0], jnp.int32))

    pltpu.sync_copy(out_v, out_hbm.at[pl.ds(wid * OUT_PER_WORKER,
                                            OUT_PER_WORKER)])


def kernel(chars, table):
    chars_flat = chars.reshape(-1)
    table_packed = jax.lax.bitcast_convert_type(
        table.astype(jnp.bfloat16).reshape(CHAR_VOCAB, PAIRS, 2),
        jnp.int32).reshape(-1)
    out_packed = pl.kernel(
        _sc_body,
        out_type=jax.ShapeDtypeStruct((NUM_WORDS * PAIRS,), jnp.int32),
        mesh=plsc.VectorSubcoreMesh(core_axis_name="c", subcore_axis_name="s"),
        compiler_params=pltpu.CompilerParams(needs_layout_passes=False),
        scratch_types=[
            pltpu.VMEM((CHAR_VOCAB * PAIRS,), jnp.int32),
            pltpu.VMEM((CHARS_PER_WORKER,), jnp.int32),
            pltpu.VMEM((OUT_PER_WORKER,), jnp.int32),
        ],
    )(chars_flat, table_packed)
    out_bf16 = jax.lax.bitcast_convert_type(
        out_packed.reshape(NUM_WORDS, PAIRS), jnp.bfloat16)
    return out_bf16.astype(jnp.float32).reshape(BATCH, MAX_WORDS, EMBED_DIM)


# speedup vs baseline: 1.0155x; 1.0155x over previous
import jax
import jax.numpy as jnp
from jax import lax
from jax.experimental import pallas as pl
from jax.experimental.pallas import tpu as pltpu
from jax.experimental.pallas import tpu_sc as plsc

CHAR_VOCAB = 1000
EMBED_DIM = 64
BATCH = 1024
MAX_WORDS = 50
MAX_CHARS = 16

PAIRS = EMBED_DIM // 2
NUM_WORDS = BATCH * MAX_WORDS
NUM_WORKERS = 32
WORDS_PER_WORKER = NUM_WORDS // NUM_WORKERS
CHARS_PER_WORKER = WORDS_PER_WORKER * MAX_CHARS
OUT_PER_WORKER = WORDS_PER_WORKER * PAIRS
DGROUPS = PAIRS // 16


def _sc_body(chars_hbm, table_hbm, out_hbm, table_v, chars_v, out_v):
    wid = lax.axis_index("s") * 2 + lax.axis_index("c")

    pltpu.sync_copy(table_hbm, table_v)
    pltpu.sync_copy(chars_hbm.at[pl.ds(wid * CHARS_PER_WORKER, CHARS_PER_WORKER)],
                    chars_v)

    lanes = lax.iota(jnp.int32, 16)
    lanes_g = [lanes + 16 * g for g in range(DGROUPS)]

    @plsc.parallel_loop(0, WORDS_PER_WORKER, unroll=4)
    def word_body(w):
        cw = w * MAX_CHARS
        row = plsc.load_gather(chars_v, [jnp.full((16,), cw, jnp.int32)]) * PAIRS
        acc = [plsc.bitcast(plsc.load_gather(table_v, [lanes_g[g] + row]),
                            jnp.bfloat16)
               for g in range(DGROUPS)]
        for c in range(1, MAX_CHARS):
            row = plsc.load_gather(
                chars_v, [jnp.full((16,), cw + c, jnp.int32)]) * PAIRS
            for g in range(DGROUPS):
                acc[g] = jnp.maximum(
                    acc[g],
                    plsc.bitcast(plsc.load_gather(table_v, [lanes_g[g] + row]),
                                 jnp.bfloat16))
        ob = w * PAIRS
        for g in range(DGROUPS):
            plsc.store_scatter(out_v, [lanes_g[g] + ob],
                               plsc.bitcast(acc[g], jnp.int32))

    pltpu.sync_copy(out_v, out_hbm.at[pl.ds(wid * OUT_PER_WORKER,
                                            OUT_PER_WORKER)])


def kernel(chars, table):
    chars_flat = chars.reshape(-1)
    table_packed = jax.lax.bitcast_convert_type(
        table.astype(jnp.bfloat16).reshape(CHAR_VOCAB, PAIRS, 2),
        jnp.int32).reshape(-1)
    out_packed = pl.kernel(
        _sc_body,
        out_type=jax.ShapeDtypeStruct((NUM_WORDS * PAIRS,), jnp.int32),
        mesh=plsc.VectorSubcoreMesh(core_axis_name="c", subcore_axis_name="s"),
        compiler_params=pltpu.CompilerParams(needs_layout_passes=False),
        scratch_types=[
            pltpu.VMEM((CHAR_VOCAB * PAIRS,), jnp.int32),
            pltpu.VMEM((CHARS_PER_WORKER,), jnp.int32),
            pltpu.VMEM((OUT_PER_WORKER,), jnp.int32),
        ],
    )(chars_flat, table_packed)
    out_bf16 = jax.lax.bitcast_convert_type(
        out_packed.reshape(NUM_WORDS, PAIRS), jnp.bfloat16)
    return out_bf16.astype(jnp.float32).reshape(BATCH, MAX_WORDS, EMBED_DIM)


# speedup vs baseline: 1.1361x; 1.1188x over previous
import jax
import jax.numpy as jnp
from jax import lax
from jax.experimental import pallas as pl
from jax.experimental.pallas import tpu as pltpu
from jax.experimental.pallas import tpu_sc as plsc

CHAR_VOCAB = 1000
EMBED_DIM = 64
BATCH = 1024
MAX_WORDS = 50
MAX_CHARS = 16

PAIRS = EMBED_DIM // 2
NUM_WORDS = BATCH * MAX_WORDS
NUM_WORKERS = 32
WORDS_PER_WORKER = NUM_WORDS // NUM_WORKERS
CHARS_PER_WORKER = WORDS_PER_WORKER * MAX_CHARS
OUT_PER_WORKER = WORDS_PER_WORKER * PAIRS
DGROUPS = PAIRS // 16


def _sc_body(chars_hbm, table_hbm, out_hbm, table_v, chars_v, out_v):
    wid = lax.axis_index("s") * 2 + lax.axis_index("c")

    pltpu.sync_copy(table_hbm, table_v)
    pltpu.sync_copy(chars_hbm.at[pl.ds(wid * CHARS_PER_WORKER, CHARS_PER_WORKER)],
                    chars_v)

    lanes = lax.iota(jnp.int32, 16)
    lanes_g = [lanes + 16 * g for g in range(DGROUPS)]

    @plsc.parallel_loop(0, WORDS_PER_WORKER, unroll=4)
    def word_body(w):
        cvec = chars_v[pl.ds(w * MAX_CHARS, MAX_CHARS)] * PAIRS
        acc = [plsc.bitcast(plsc.load_gather(table_v, [lanes_g[g] + cvec[0]]),
                            jnp.bfloat16)
               for g in range(DGROUPS)]
        for c in range(1, MAX_CHARS):
            row = cvec[c]
            for g in range(DGROUPS):
                acc[g] = jnp.maximum(
                    acc[g],
                    plsc.bitcast(plsc.load_gather(table_v, [lanes_g[g] + row]),
                                 jnp.bfloat16))
        ob = w * PAIRS
        for g in range(DGROUPS):
            plsc.store_scatter(out_v, [lanes_g[g] + ob],
                               plsc.bitcast(acc[g], jnp.int32))

    pltpu.sync_copy(out_v, out_hbm.at[pl.ds(wid * OUT_PER_WORKER,
                                            OUT_PER_WORKER)])


def kernel(chars, table):
    chars_flat = chars.reshape(-1)
    table_packed = jax.lax.bitcast_convert_type(
        table.astype(jnp.bfloat16).reshape(CHAR_VOCAB, PAIRS, 2),
        jnp.int32).reshape(-1)
    out_packed = pl.kernel(
        _sc_body,
        out_type=jax.ShapeDtypeStruct((NUM_WORDS * PAIRS,), jnp.int32),
        mesh=plsc.VectorSubcoreMesh(core_axis_name="c", subcore_axis_name="s"),
        compiler_params=pltpu.CompilerParams(needs_layout_passes=False),
        scratch_types=[
            pltpu.VMEM((CHAR_VOCAB * PAIRS,), jnp.int32),
            pltpu.VMEM((CHARS_PER_WORKER,), jnp.int32),
            pltpu.VMEM((OUT_PER_WORKER,), jnp.int32),
        ],
    )(chars_flat, table_packed)
    out_bf16 = jax.lax.bitcast_convert_type(
        out_packed.reshape(NUM_WORDS, PAIRS), jnp.bfloat16)
    return out_bf16.astype(jnp.float32).reshape(BATCH, MAX_WORDS, EMBED_DIM)


# speedup vs baseline: 1.4529x; 1.2788x over previous
import jax
import jax.numpy as jnp
from jax import lax
from jax.experimental import pallas as pl
from jax.experimental.pallas import tpu as pltpu
from jax.experimental.pallas import tpu_sc as plsc

CHAR_VOCAB = 1000
EMBED_DIM = 64
BATCH = 1024
MAX_WORDS = 50
MAX_CHARS = 16

PAIRS = EMBED_DIM // 2
NUM_WORDS = BATCH * MAX_WORDS
NUM_WORKERS = 32
WORDS_PER_WORKER = NUM_WORDS // NUM_WORKERS
CHARS_PER_WORKER = WORDS_PER_WORKER * MAX_CHARS
OUT_PER_WORKER = WORDS_PER_WORKER * EMBED_DIM
DGROUPS = PAIRS // 16

PACK_ROWS = 200
PACK_CHUNKS = CHAR_VOCAB // PACK_ROWS
OUT_PASSES = 2
WORDS_PER_PASS = WORDS_PER_WORKER // OUT_PASSES
OUT_PER_PASS = WORDS_PER_PASS * EMBED_DIM

_HI_MASK = jnp.int32(-65536)
_ROUND = jnp.int32(0x8000)


def _sc_body(chars_hbm, table_hbm, out_hbm, stage_v, packed_v, chars_v, out_v):
    wid = lax.axis_index("s") * 2 + lax.axis_index("c")

    pltpu.sync_copy(chars_hbm.at[pl.ds(wid * CHARS_PER_WORKER, CHARS_PER_WORKER)],
                    chars_v)

    lanes = lax.iota(jnp.int32, 16)
    lanes_g = [lanes + 16 * g for g in range(DGROUPS)]

    for chunk in range(PACK_CHUNKS):
        pltpu.sync_copy(
            table_hbm.at[pl.ds(chunk * PACK_ROWS * EMBED_DIM,
                               PACK_ROWS * EMBED_DIM)],
            stage_v)
        pbase = chunk * PACK_ROWS * PAIRS

        @plsc.parallel_loop(0, PACK_ROWS, unroll=4)
        def pack_body(r):
            for g in range(DGROUPS):
                a = plsc.load_gather(stage_v, [lanes_g[g] + r * EMBED_DIM])
                b = plsc.load_gather(stage_v,
                                     [lanes_g[g] + (r * EMBED_DIM + PAIRS)])
                lo = lax.shift_right_logical(a + _ROUND, 16)
                hi = (b + _ROUND) & _HI_MASK
                plsc.store_scatter(packed_v, [lanes_g[g] + (r * PAIRS + pbase)],
                                   lo | hi)

    for half in range(OUT_PASSES):
        cbase = half * WORDS_PER_PASS * MAX_CHARS

        @plsc.parallel_loop(0, WORDS_PER_PASS, unroll=4)
        def word_body(w):
            cvec = chars_v[pl.ds(w * MAX_CHARS + cbase, MAX_CHARS)] * PAIRS
            acc = [plsc.bitcast(
                       plsc.load_gather(packed_v, [lanes_g[g] + cvec[0]]),
                       jnp.bfloat16)
                   for g in range(DGROUPS)]
            for c in range(1, MAX_CHARS):
                row = cvec[c]
                for g in range(DGROUPS):
                    acc[g] = jnp.maximum(
                        acc[g],
                        plsc.bitcast(
                            plsc.load_gather(packed_v, [lanes_g[g] + row]),
                            jnp.bfloat16))
            ob = w * EMBED_DIM
            for g in range(DGROUPS):
                s = plsc.bitcast(acc[g], jnp.int32)
                plsc.store_scatter(out_v, [lanes_g[g] + ob], s << 16)
                plsc.store_scatter(out_v, [lanes_g[g] + (ob + PAIRS)],
                                   s & _HI_MASK)

        pltpu.sync_copy(out_v,
                        out_hbm.at[pl.ds(wid * OUT_PER_WORKER
                                         + half * OUT_PER_PASS, OUT_PER_PASS)])


def kernel(chars, table):
    chars_flat = chars.reshape(-1)
    table_bits = jax.lax.bitcast_convert_type(table, jnp.int32).reshape(-1)
    out_bits = pl.kernel(
        _sc_body,
        out_type=jax.ShapeDtypeStruct((NUM_WORDS * EMBED_DIM,), jnp.int32),
        mesh=plsc.VectorSubcoreMesh(core_axis_name="c", subcore_axis_name="s"),
        compiler_params=pltpu.CompilerParams(needs_layout_passes=False),
        scratch_types=[
            pltpu.VMEM((PACK_ROWS * EMBED_DIM,), jnp.int32),
            pltpu.VMEM((CHAR_VOCAB * PAIRS,), jnp.int32),
            pltpu.VMEM((CHARS_PER_WORKER,), jnp.int32),
            pltpu.VMEM((OUT_PER_PASS,), jnp.int32),
        ],
    )(chars_flat, table_bits)
    return jax.lax.bitcast_convert_type(out_bits, jnp.float32).reshape(
        BATCH, MAX_WORDS, EMBED_DIM)


# speedup vs baseline: 1.5460x; 1.0641x over previous
import jax
import jax.numpy as jnp
from jax import lax
from jax.experimental import pallas as pl
from jax.experimental.pallas import tpu as pltpu
from jax.experimental.pallas import tpu_sc as plsc

CHAR_VOCAB = 1000
EMBED_DIM = 64
BATCH = 1024
MAX_WORDS = 50
MAX_CHARS = 16

PAIRS = EMBED_DIM // 2
NUM_WORDS = BATCH * MAX_WORDS
NUM_WORKERS = 32
WORDS_PER_WORKER = NUM_WORDS // NUM_WORKERS
CHARS_PER_WORKER = WORDS_PER_WORKER * MAX_CHARS
OUT_PER_WORKER = WORDS_PER_WORKER * EMBED_DIM
DGROUPS = PAIRS // 16

PACK_ROWS = 200
PACK_CHUNKS = CHAR_VOCAB // PACK_ROWS
OUT_PASSES = 2
WORDS_PER_PASS = WORDS_PER_WORKER // OUT_PASSES
OUT_PER_PASS = WORDS_PER_PASS * EMBED_DIM

_HI_MASK = jnp.int32(-65536)
_ROUND = jnp.int32(0x8000)


def _sc_body(chars_hbm, table_hbm, out_hbm, stage_v, packed_v, chars_v, out_v):
    wid = lax.axis_index("s") * 2 + lax.axis_index("c")

    pltpu.sync_copy(chars_hbm.at[pl.ds(wid * CHARS_PER_WORKER, CHARS_PER_WORKER)],
                    chars_v)

    lanes = lax.iota(jnp.int32, 16)
    lanes_g = [lanes + 16 * g for g in range(DGROUPS)]

    for chunk in range(PACK_CHUNKS):
        pltpu.sync_copy(
            table_hbm.at[pl.ds(chunk * PACK_ROWS * EMBED_DIM,
                               PACK_ROWS * EMBED_DIM)],
            stage_v)
        pbase = chunk * PACK_ROWS * PAIRS

        @plsc.parallel_loop(0, PACK_ROWS, unroll=4)
        def pack_body(r):
            for g in range(DGROUPS):
                a = plsc.bitcast(
                    plsc.load_gather(stage_v, [lanes_g[g] + r * EMBED_DIM]),
                    jnp.int32)
                b = plsc.bitcast(
                    plsc.load_gather(stage_v,
                                     [lanes_g[g] + (r * EMBED_DIM + PAIRS)]),
                    jnp.int32)
                lo = lax.shift_right_logical(a + _ROUND, 16)
                hi = (b + _ROUND) & _HI_MASK
                plsc.store_scatter(packed_v, [lanes_g[g] + (r * PAIRS + pbase)],
                                   lo | hi)

    for half in range(OUT_PASSES):
        cbase = half * WORDS_PER_PASS * MAX_CHARS

        @plsc.parallel_loop(0, WORDS_PER_PASS, unroll=4)
        def word_body(w):
            cvec = chars_v[pl.ds(w * MAX_CHARS + cbase, MAX_CHARS)] * PAIRS
            acc = [plsc.bitcast(
                       plsc.load_gather(packed_v, [lanes_g[g] + cvec[0]]),
                       jnp.bfloat16)
                   for g in range(DGROUPS)]
            for c in range(1, MAX_CHARS):
                row = cvec[c]
                for g in range(DGROUPS):
                    acc[g] = jnp.maximum(
                        acc[g],
                        plsc.bitcast(
                            plsc.load_gather(packed_v, [lanes_g[g] + row]),
                            jnp.bfloat16))
            ob = w * EMBED_DIM
            for g in range(DGROUPS):
                s = plsc.bitcast(acc[g], jnp.int32)
                plsc.store_scatter(out_v, [lanes_g[g] + ob],
                                   plsc.bitcast(s << 16, jnp.float32))
                plsc.store_scatter(out_v, [lanes_g[g] + (ob + PAIRS)],
                                   plsc.bitcast(s & _HI_MASK, jnp.float32))

        pltpu.sync_copy(out_v,
                        out_hbm.at[pl.ds(wid * OUT_PER_WORKER
                                         + half * OUT_PER_PASS, OUT_PER_PASS)])


def kernel(chars, table):
    out = pl.kernel(
        _sc_body,
        out_type=jax.ShapeDtypeStruct((NUM_WORDS * EMBED_DIM,), jnp.float32),
        mesh=plsc.VectorSubcoreMesh(core_axis_name="c", subcore_axis_name="s"),
        compiler_params=pltpu.CompilerParams(needs_layout_passes=False),
        scratch_types=[
            pltpu.VMEM((PACK_ROWS * EMBED_DIM,), jnp.float32),
            pltpu.VMEM((CHAR_VOCAB * PAIRS,), jnp.int32),
            pltpu.VMEM((CHARS_PER_WORKER,), jnp.int32),
            pltpu.VMEM((OUT_PER_PASS,), jnp.float32),
        ],
    )(chars.reshape(-1), table.reshape(-1))
    return out.reshape(BATCH, MAX_WORDS, EMBED_DIM)
